# baseline (speedup 1.0000x reference)
import jax
import jax.numpy as jnp
from jax import lax
from jax.experimental import pallas as pl
from jax.experimental.pallas import tpu as pltpu

B = 64
NB = 30
NSEM = 11
D = 256
GB = 16
RB = GB * NB
F32 = jnp.float32


def _lot_body(z_ref, w_ref, b_ref, out_ref):
    acc = jnp.dot(z_ref[:, :], w_ref[:, :], preferred_element_type=F32)
    out_ref[:, :] = jnp.maximum(acc + b_ref[:, :], 0.0)


def _fused_body(lot2_ref, lot3_ref,
                bw1_ref, bb1_ref, bw2_ref, bb2_ref,
                w1_ref, p1_ref, w2_ref, p2_ref, edw_ref, edb_ref,
                w1il_ref, w1ip_ref, w1jl_ref, w1jp_ref, b1_ref,
                w2i_ref, w2j_ref, b2_ref, w3i_ref, w3j_ref, b3_ref,
                nedw_ref, nedb_ref, lhw_ref, lhb_ref, bhw_ref, bhb_ref,
                bdw_ref, bdb_ref, mhw_ref, mhb_ref,
                ar_ref, adj_ref, lep_ref, lg_ref, lu_ref, lb_ref, lm_ref,
                s_ref, nbr_ref, x30_ref, af0_ref, apad_ref, bpad_ref):
    j = pl.program_id(0)
    lot = lot2_ref[:, :]

    h = jnp.maximum(jnp.dot(lot, bw1_ref[:, :], preferred_element_type=F32)
                    + bb1_ref[:, :], 0.0)
    ar_ref[:, :] = jnp.dot(h, bw2_ref[:, :], preferred_element_type=F32) + bb2_ref[:, :]

    A = jnp.dot(lot, w1_ref[:, :], preferred_element_type=F32) + p1_ref[:, :]
    Bt = jnp.dot(lot, w2_ref[:, :], preferred_element_type=F32) + p2_ref[:, :]
    for i in range(GB):
        apad_ref[32 * i:32 * i + NB, :] = A[NB * i:NB * i + NB, :]
        bpad_ref[32 * i:32 * i + NB, :] = Bt[NB * i:NB * i + NB, :]

    w = edw_ref[:, :][None, :, :]
    eb = edb_ref[0, 0]
    for i in range(GB):
        Ai = apad_ref[32 * i:32 * i + NB, :]
        Bi = bpad_ref[32 * i:32 * i + NB, :]
        T = jnp.maximum(Ai[:, None, :] + Bi[None, :, :], 0.0)
        adj_ref[i, :, :] = jnp.sum(T * w, axis=-1) + eb

    adjb = jax.nn.sigmoid(adj_ref[:, :, :])
    adj_ref[:, :, :] = adjb
    af = (adjb >= 0.5).astype(F32)
    s_acc = jnp.sum(af, axis=0)
    nbr_acc = None
    for i in range(GB):
        nbr = lax.dot_general(af[i], lot3_ref[i, :, :], (((0,), (0,)), ((), ())),
                              preferred_element_type=F32)
        nbr_acc = nbr if nbr_acc is None else nbr_acc + nbr

    @pl.when(j == 0)
    def _init():
        s_ref[:, :] = s_acc
        nbr_ref[:, :] = nbr_acc
        x30_ref[:, :] = lot3_ref[0, :, :]
        af0_ref[:, :] = af[0]

    @pl.when(j > 0)
    def _acc():
        s_ref[:, :] = s_ref[:, :] + s_acc
        nbr_ref[:, :] = nbr_ref[:, :] + nbr_acc

    @pl.when(j == pl.num_programs(0) - 1)
    def _mp():
        S = s_ref[:, :]
        ones = jnp.ones((NB, 1), F32)
        deg = lax.dot_general(S, ones, (((0,), (0,)), ((), ())),
                              preferred_element_type=F32)
        invd = 1.0 / jnp.where(deg > 0, deg, 1.0)
        mask = deg > 0

        nbrm = nbr_ref[:, :] * invd
        sm = lax.dot_general(S, w1jp_ref[:, :], (((0,), (0,)), ((), ())),
                             preferred_element_type=F32) * invd
        out1 = (jnp.dot(x30_ref[:, :], w1il_ref[:, :], preferred_element_type=F32)
                + w1ip_ref[:, :]
                + jnp.dot(nbrm, w1jl_ref[:, :], preferred_element_type=F32)
                + sm + b1_ref[:, :])
        d = jnp.maximum(jnp.where(mask, out1, 0.0), 0.0)

        af0 = af0_ref[:, :]
        for wi_ref, wj_ref, bb_ref in ((w2i_ref, w2j_ref, b2_ref),
                                       (w3i_ref, w3j_ref, b3_ref)):
            nbr2 = lax.dot_general(af0, d, (((0,), (0,)), ((), ())),
                                   preferred_element_type=F32) * invd
            out = (jnp.dot(d, wi_ref[:, :], preferred_element_type=F32)
                   + jnp.dot(nbr2, wj_ref[:, :], preferred_element_type=F32)
                   + bb_ref[:, :])
            d = jnp.maximum(jnp.where(mask, out, 0.0), 0.0)

        def head(w_ref, b_ref):
            return jnp.dot(d, w_ref[:, :], preferred_element_type=F32) + b_ref[:, :]

        def softmax(x):
            m = jnp.max(x, axis=-1, keepdims=True)
            e = jnp.exp(x - m)
            return e / jnp.sum(e, axis=-1, keepdims=True)

        rest = B * NB - NB
        lep_ref[:NB, :] = jax.nn.sigmoid(head(nedw_ref, nedb_ref))
        lep_ref[NB:, :] = jnp.broadcast_to(jax.nn.sigmoid(nedb_ref[:, :]), (rest, 1))
        lu_ref[:NB, :] = softmax(head(lhw_ref, lhb_ref))
        lu_ref[NB:, :] = jnp.broadcast_to(softmax(lhb_ref[:, :]), (rest, NSEM))
        lg_ref[:NB, :] = head(bhw_ref, bhb_ref)
        lg_ref[NB:, :] = jnp.broadcast_to(bhb_ref[:, :], (rest, 5))
        lb_ref[:NB, :] = head(bdw_ref, bdb_ref)
        lb_ref[NB:, :] = jnp.broadcast_to(bdb_ref[:, :], (rest, 4))
        lm_ref[:NB, :] = head(mhw_ref, mhb_ref)
        lm_ref[NB:, :] = jnp.broadcast_to(mhb_ref[:, :], (rest, 2))


def kernel(z, lid_W, lid_b, bbd1_W, bbd1_b, bbd2_W, bbd2_b, ed1_W, ed1_b,
           ed2_W, ed2_b, mp1_W, mp1_b, mp2_W, mp2_b, mp3_W, mp3_b,
           ned_W, ned_b, lh_W, lh_b, bh_W, bh_b, bdh_W, bdh_b, mh_W, mh_b):
    NBLK = 2
    BN = (NB * D) // NBLK
    lot2d = pl.pallas_call(
        _lot_body,
        grid=(NBLK,),
        in_specs=[
            pl.BlockSpec((B, D), lambda j: (0, 0)),
            pl.BlockSpec((D, BN), lambda j: (0, j)),
            pl.BlockSpec((1, BN), lambda j: (0, j)),
        ],
        out_specs=pl.BlockSpec((B, BN), lambda j: (0, j)),
        out_shape=jax.ShapeDtypeStruct((B, NB * D), F32),
    )(z, lid_W, lid_b.reshape(1, NB * D))
    lot = lot2d.reshape(B * NB, D)
    lot3 = lot2d.reshape(B, NB, D)

    W1l, W1p = ed1_W[:D], ed1_W[D:D + NB]
    W2l, W2p = ed1_W[D + NB:2 * D + NB], ed1_W[2 * D + NB:]
    pos1 = jnp.tile(W1p, (GB, 1)) + ed1_b[None, :]
    pos2 = jnp.tile(W2p, (GB, 1))
    cmap2 = lambda *s: pl.BlockSpec(s, lambda j: (0,) * len(s))
    ar, adj, lep, lg, lu, lb, lm = pl.pallas_call(
        _fused_body,
        grid=(B // GB,),
        in_specs=[
            pl.BlockSpec((RB, D), lambda j: (j, 0)),
            pl.BlockSpec((GB, NB, D), lambda j: (j, 0, 0)),
            cmap2(D, D), cmap2(1, D), cmap2(D, 1), cmap2(1, 1),
            cmap2(D, D), cmap2(RB, D), cmap2(D, D), cmap2(RB, D),
            cmap2(1, D), cmap2(1, 1),
            cmap2(D, D), cmap2(NB, D), cmap2(D, D), cmap2(NB, D), cmap2(1, D),
            cmap2(D, D), cmap2(D, D), cmap2(1, D),
            cmap2(D, D), cmap2(D, D), cmap2(1, D),
            cmap2(D, 1), cmap2(1, 1), cmap2(D, NSEM), cmap2(1, NSEM),
            cmap2(D, 5), cmap2(1, 5), cmap2(D, 4), cmap2(1, 4),
            cmap2(D, 2), cmap2(1, 2),
        ],
        out_specs=[
            pl.BlockSpec((RB, 1), lambda j: (j, 0)),
            pl.BlockSpec((GB, NB, NB), lambda j: (j, 0, 0)),
            cmap2(B * NB, 1), cmap2(B * NB, 5), cmap2(B * NB, NSEM),
            cmap2(B * NB, 4), cmap2(B * NB, 2),
        ],
        out_shape=[
            jax.ShapeDtypeStruct((B * NB, 1), F32),
            jax.ShapeDtypeStruct((B, NB, NB), F32),
            jax.ShapeDtypeStruct((B * NB, 1), F32),
            jax.ShapeDtypeStruct((B * NB, 5), F32),
            jax.ShapeDtypeStruct((B * NB, NSEM), F32),
            jax.ShapeDtypeStruct((B * NB, 4), F32),
            jax.ShapeDtypeStruct((B * NB, 2), F32),
        ],
        scratch_shapes=[
            pltpu.VMEM((NB, NB), F32),
            pltpu.VMEM((NB, D), F32),
            pltpu.VMEM((NB, D), F32),
            pltpu.VMEM((NB, NB), F32),
            pltpu.VMEM((GB * 32, D), F32),
            pltpu.VMEM((GB * 32, D), F32),
        ],
    )(lot, lot3,
      bbd1_W, bbd1_b.reshape(1, D), bbd2_W, bbd2_b.reshape(1, 1),
      W1l, pos1, W2l, pos2, ed2_W.reshape(1, D), ed2_b.reshape(1, 1),
      mp1_W[:D], mp1_W[D:D + NB], mp1_W[D + NB:2 * D + NB], mp1_W[2 * D + NB:],
      mp1_b.reshape(1, D),
      mp2_W[:D], mp2_W[D:], mp2_b.reshape(1, D),
      mp3_W[:D], mp3_W[D:], mp3_b.reshape(1, D),
      ned_W, ned_b.reshape(1, 1), lh_W, lh_b.reshape(1, NSEM),
      bh_W, bh_b.reshape(1, 5), bdh_W, bdh_b.reshape(1, 4),
      mh_W, mh_b.reshape(1, 2))

    return (lep, lg, lu, lb, lm, adj, ar)
